# baseline (speedup 1.0000x reference)
import functools

import jax
import jax.numpy as jnp
from jax import lax
from jax.experimental import pallas as pl
from jax.experimental.pallas import tpu as pltpu
from jax.experimental.pallas import tpu_sc as plsc

_B = 128
_CHUNK = 32


def _sc_row_gather(src, idx, n_out):
    info = plsc.get_sparse_core_info()
    nw = info.num_cores * info.num_subcores
    d = src.shape[1]
    assert n_out % (nw * 2 * _CHUNK) == 0
    per_w = n_out // nw
    n_pairs = per_w // (2 * _CHUNK)

    mesh = plsc.VectorSubcoreMesh(core_axis_name="c", subcore_axis_name="s")

    @functools.partial(
        pl.kernel,
        mesh=mesh,
        out_type=jax.ShapeDtypeStruct((n_out, d), src.dtype),
        scratch_types=[
            pltpu.VMEM((per_w,), jnp.int32),
            pltpu.VMEM((2, _CHUNK, d), src.dtype),
            pltpu.SemaphoreType.DMA,
            pltpu.SemaphoreType.DMA,
            pltpu.SemaphoreType.DMA,
            pltpu.SemaphoreType.DMA,
        ],
    )
    def k(src_hbm, idx_hbm, out_hbm, idx_v, rows_v, g0, g1, w0, w1):
        wid = lax.axis_index("s") * info.num_cores + lax.axis_index("c")
        base0 = wid * per_w
        pltpu.sync_copy(idx_hbm.at[pl.ds(base0, per_w)], idx_v)
        gsem = (g0, g1)
        wsem = (w0, w1)

        def gather(c, b):
            return pltpu.make_async_copy(
                src_hbm.at[idx_v.at[pl.ds(c * _CHUNK, _CHUNK)]],
                rows_v.at[b], gsem[b])

        def write(c, b):
            return pltpu.make_async_copy(
                rows_v.at[b], out_hbm.at[pl.ds(base0 + c * _CHUNK, _CHUNK)],
                wsem[b])

        def body(p, _):
            for b in (0, 1):
                c = p * 2 + b

                @pl.when(p > 0)
                def _():
                    write(c, b).wait()

                gather(c, b).start()
            for b in (0, 1):
                c = p * 2 + b
                gather(c, b).wait()
                write(c, b).start()
            return ()

        lax.fori_loop(0, n_pairs, body, (), unroll=False)
        for b in (0, 1):
            write((n_pairs - 1) * 2 + b, b).wait()

    return k(src, idx)


def _mm_kernel(b2e_ref, nused_ref, x_ref, w1_ref, w2_ref, ws_ref, o_ref):
    i = pl.program_id(0)

    @pl.when(i < nused_ref[0])
    def _():
        xb = x_ref[...]
        w1 = w1_ref[0]
        h = lax.dot_general(xb, w1, (((1,), (1,)), ((), ())),
                            preferred_element_type=jnp.float32)
        f = w1.shape[0] // 2
        y = h[:, :f]
        g = h[:, f:]
        act = (y * (g * jax.nn.sigmoid(g))).astype(jnp.bfloat16)
        w2 = w2_ref[0]
        ob = lax.dot_general(act, w2, (((1,), (1,)), ((), ())),
                             preferred_element_type=jnp.float32)
        o_ref[...] = (ob * ws_ref[...]).astype(jnp.bfloat16)


def _grouped_mlp(x_sorted, fc1, fc2, w_sorted, b2e, num_used, num_blocks):
    p_max, d_model = x_sorted.shape
    e, two_ff, _ = fc1.shape
    d_ff = two_ff // 2

    def last_used(i, nu):
        return jnp.minimum(i, nu[0] - 1)

    grid_spec = pltpu.PrefetchScalarGridSpec(
        num_scalar_prefetch=2,
        grid=(num_blocks,),
        in_specs=[
            pl.BlockSpec((_B, d_model), lambda i, be, nu: (last_used(i, nu), 0)),
            pl.BlockSpec((1, two_ff, d_model),
                         lambda i, be, nu: (be[last_used(i, nu)], 0, 0)),
            pl.BlockSpec((1, d_model, d_ff),
                         lambda i, be, nu: (be[last_used(i, nu)], 0, 0)),
            pl.BlockSpec((_B, 1), lambda i, be, nu: (last_used(i, nu), 0)),
        ],
        out_specs=pl.BlockSpec((_B, d_model),
                               lambda i, be, nu: (last_used(i, nu), 0)),
    )
    return pl.pallas_call(
        _mm_kernel,
        grid_spec=grid_spec,
        out_shape=jax.ShapeDtypeStruct((p_max, d_model), jnp.bfloat16),
        compiler_params=pltpu.CompilerParams(
            dimension_semantics=("arbitrary",),
            vmem_limit_bytes=100 * 1024 * 1024,
        ),
    )(b2e, num_used, x_sorted, fc1, fc2, w_sorted)


def _to_u32(a_bf16):
    t, d = a_bf16.shape
    return lax.bitcast_convert_type(
        a_bf16.reshape(t, d // 2, 2), jnp.uint32)


def _from_u32(a_u32):
    t, d2 = a_u32.shape
    return lax.bitcast_convert_type(a_u32, jnp.bfloat16).reshape(t, d2 * 2)


def kernel(x, weights, indices, fc1_weights, fc2_weights):
    t, d_model = x.shape
    e = fc1_weights.shape[0]
    p_max = t + e * _B
    num_blocks = t // _B + e

    flat = indices.reshape(-1).astype(jnp.int32)
    onehot = (flat[:, None] == jnp.arange(e, dtype=jnp.int32)[None, :])
    csum = jnp.cumsum(onehot.astype(jnp.int32), axis=0)
    counts = csum[-1]
    rank = jnp.take_along_axis(csum, flat[:, None], axis=1)[:, 0] - 1
    nblk = (counts + _B - 1) // _B
    starts = jnp.concatenate(
        [jnp.zeros((1,), jnp.int32), jnp.cumsum(nblk * _B).astype(jnp.int32)]
    )
    num_used = (starts[e] // _B).reshape(1).astype(jnp.int32)

    pos = (jnp.take(starts, flat) + rank).astype(jnp.int32)
    src = jnp.zeros((p_max,), jnp.int32).at[pos].set(
        jnp.arange(t, dtype=jnp.int32))

    blk_p = jnp.arange(num_blocks, dtype=jnp.int32) * _B
    b2e = jnp.minimum(
        jnp.searchsorted(starts[1:], blk_p, side="right").astype(jnp.int32),
        e - 1)

    w_sorted = jnp.zeros((p_max,), jnp.float32).at[pos].set(
        weights[:, 0]).reshape(p_max, 1)

    x_u32 = _to_u32(x.astype(jnp.bfloat16))
    xs_u32 = _sc_row_gather(x_u32, src, p_max)
    x_sorted = _from_u32(xs_u32)
    y_sorted = _grouped_mlp(x_sorted, fc1_weights.astype(jnp.bfloat16),
                            fc2_weights.astype(jnp.bfloat16), w_sorted,
                            b2e, num_used, num_blocks)
    out_u32 = _sc_row_gather(_to_u32(y_sorted), pos, t)
    return _from_u32(out_u32).astype(jnp.float32)


# speedup vs baseline: 2.5430x; 2.5430x over previous
import functools

import jax
import jax.numpy as jnp
from jax import lax
from jax.experimental import pallas as pl
from jax.experimental.pallas import tpu as pltpu
from jax.experimental.pallas import tpu_sc as plsc

_B = 128
_CHUNK = 16


def _sc_row_gather(src, idx, n_out):
    info = plsc.get_sparse_core_info()
    nw = info.num_cores * info.num_subcores
    d = src.shape[1]
    assert n_out % (nw * 2 * _CHUNK) == 0
    per_w = n_out // nw
    n_pairs = per_w // (2 * _CHUNK)

    mesh = plsc.VectorSubcoreMesh(core_axis_name="c", subcore_axis_name="s")

    @functools.partial(
        pl.kernel,
        mesh=mesh,
        out_type=jax.ShapeDtypeStruct((n_out, d), src.dtype),
        scratch_types=[
            pltpu.VMEM((per_w,), jnp.int32),
            pltpu.VMEM((2, _CHUNK, d), src.dtype),
            pltpu.SemaphoreType.DMA,
            pltpu.SemaphoreType.DMA,
            pltpu.SemaphoreType.DMA,
            pltpu.SemaphoreType.DMA,
        ],
    )
    def k(src_hbm, idx_hbm, out_hbm, idx_v, rows_v, g0, g1, w0, w1):
        wid = lax.axis_index("s") * info.num_cores + lax.axis_index("c")
        base0 = wid * per_w
        pltpu.sync_copy(idx_hbm.at[pl.ds(base0, per_w)], idx_v)
        gsem = (g0, g1)
        wsem = (w0, w1)

        def gather(c, b):
            return pltpu.make_async_copy(
                src_hbm.at[idx_v.at[pl.ds(c * _CHUNK, _CHUNK)]],
                rows_v.at[b], gsem[b])

        def write(c, b):
            return pltpu.make_async_copy(
                rows_v.at[b], out_hbm.at[pl.ds(base0 + c * _CHUNK, _CHUNK)],
                wsem[b])

        def body(p, _):
            for b in (0, 1):
                c = p * 2 + b

                @pl.when(p > 0)
                def _():
                    write(c, b).wait()

                gather(c, b).start()
            for b in (0, 1):
                c = p * 2 + b
                gather(c, b).wait()
                write(c, b).start()
            return ()

        lax.fori_loop(0, n_pairs, body, (), unroll=False)
        for b in (0, 1):
            write((n_pairs - 1) * 2 + b, b).wait()

    return k(src, idx)


def _mm_kernel(b2e_ref, nused_ref, x_ref, w1_ref, w2_ref, ws_ref, o_ref):
    i = pl.program_id(0)

    @pl.when(i < nused_ref[0])
    def _():
        xb = x_ref[...].astype(jnp.bfloat16)
        w1 = w1_ref[0]
        h = lax.dot_general(xb, w1, (((1,), (1,)), ((), ())),
                            preferred_element_type=jnp.float32)
        f = w1.shape[0] // 2
        y = h[:, :f]
        g = h[:, f:]
        act = (y * (g * jax.nn.sigmoid(g))).astype(jnp.bfloat16)
        w2 = w2_ref[0]
        ob = lax.dot_general(act, w2, (((1,), (1,)), ((), ())),
                             preferred_element_type=jnp.float32)
        o_ref[...] = ob * ws_ref[...]


def _grouped_mlp(x_sorted, fc1, fc2, w_sorted, b2e, num_used, num_blocks):
    p_max, d_model = x_sorted.shape
    e, two_ff, _ = fc1.shape
    d_ff = two_ff // 2

    def last_used(i, nu):
        return jnp.minimum(i, nu[0] - 1)

    grid_spec = pltpu.PrefetchScalarGridSpec(
        num_scalar_prefetch=2,
        grid=(num_blocks,),
        in_specs=[
            pl.BlockSpec((_B, d_model), lambda i, be, nu: (last_used(i, nu), 0)),
            pl.BlockSpec((1, two_ff, d_model),
                         lambda i, be, nu: (be[last_used(i, nu)], 0, 0)),
            pl.BlockSpec((1, d_model, d_ff),
                         lambda i, be, nu: (be[last_used(i, nu)], 0, 0)),
            pl.BlockSpec((_B, 1), lambda i, be, nu: (last_used(i, nu), 0)),
        ],
        out_specs=pl.BlockSpec((_B, d_model),
                               lambda i, be, nu: (last_used(i, nu), 0)),
    )
    return pl.pallas_call(
        _mm_kernel,
        grid_spec=grid_spec,
        out_shape=jax.ShapeDtypeStruct((p_max, d_model), jnp.float32),
        compiler_params=pltpu.CompilerParams(
            dimension_semantics=("arbitrary",),
            vmem_limit_bytes=100 * 1024 * 1024,
        ),
    )(b2e, num_used, x_sorted, fc1, fc2, w_sorted)


def kernel(x, weights, indices, fc1_weights, fc2_weights):
    t, d_model = x.shape
    e = fc1_weights.shape[0]
    p_max = t + e * _B
    num_blocks = t // _B + e

    flat = indices.reshape(-1).astype(jnp.int32)
    sort_ids = jnp.argsort(flat, stable=True).astype(jnp.int32)
    sorted_flat = jnp.take(flat, sort_ids)
    bounds = jnp.searchsorted(
        sorted_flat, jnp.arange(e + 1, dtype=jnp.int32), side="left"
    ).astype(jnp.int32)
    counts = bounds[1:] - bounds[:-1]
    nblk = (counts + _B - 1) // _B
    starts = jnp.concatenate(
        [jnp.zeros((1,), jnp.int32), jnp.cumsum(nblk * _B).astype(jnp.int32)]
    )
    num_used = (starts[e] // _B).reshape(1).astype(jnp.int32)

    p = jnp.arange(p_max, dtype=jnp.int32)
    e_of_p = jnp.minimum(
        jnp.searchsorted(starts[1:], p, side="right").astype(jnp.int32), e - 1)
    local = p - jnp.take(starts, e_of_p)
    r = jnp.take(bounds, e_of_p) + local
    valid = local < jnp.take(counts, e_of_p)
    src = jnp.where(valid, jnp.take(sort_ids, jnp.clip(r, 0, t - 1)),
                    p & (t - 1))
    src = src.astype(jnp.int32)

    inv_rank = jnp.argsort(sort_ids).astype(jnp.int32)
    pos = (jnp.take(starts, flat) + (inv_rank - jnp.take(bounds, flat))
           ).astype(jnp.int32)

    blk_p = jnp.arange(num_blocks, dtype=jnp.int32) * _B
    b2e = jnp.minimum(
        jnp.searchsorted(starts[1:], blk_p, side="right").astype(jnp.int32),
        e - 1)

    w_sorted = jnp.take(weights[:, 0], src).reshape(p_max, 1)

    x_sorted = _sc_row_gather(x, src, p_max)
    y_sorted = _grouped_mlp(x_sorted, fc1_weights.astype(jnp.bfloat16),
                            fc2_weights.astype(jnp.bfloat16), w_sorted,
                            b2e, num_used, num_blocks)
    out = _sc_row_gather(y_sorted, pos, t)
    return out


# speedup vs baseline: 3.1422x; 1.2356x over previous
import functools

import jax
import jax.numpy as jnp
from jax import lax
from jax.experimental import pallas as pl
from jax.experimental.pallas import tpu as pltpu
from jax.experimental.pallas import tpu_sc as plsc

_B = 128
_CHUNK = 16


def _sc_row_gather(src, idx, n_out):
    info = plsc.get_sparse_core_info()
    nw = info.num_cores * info.num_subcores
    d = src.shape[1]
    assert n_out % (nw * 2 * _CHUNK) == 0
    per_w = n_out // nw
    n_pairs = per_w // (2 * _CHUNK)

    mesh = plsc.VectorSubcoreMesh(core_axis_name="c", subcore_axis_name="s")

    @functools.partial(
        pl.kernel,
        mesh=mesh,
        out_type=jax.ShapeDtypeStruct((n_out, d), src.dtype),
        scratch_types=[
            pltpu.VMEM((per_w,), jnp.int32),
            pltpu.VMEM((2, _CHUNK, d), src.dtype),
            pltpu.SemaphoreType.DMA,
            pltpu.SemaphoreType.DMA,
            pltpu.SemaphoreType.DMA,
            pltpu.SemaphoreType.DMA,
        ],
    )
    def k(src_hbm, idx_hbm, out_hbm, idx_v, rows_v, g0, g1, w0, w1):
        wid = lax.axis_index("s") * info.num_cores + lax.axis_index("c")
        base0 = wid * per_w
        pltpu.sync_copy(idx_hbm.at[pl.ds(base0, per_w)], idx_v)
        gsem = (g0, g1)
        wsem = (w0, w1)

        def gather(c, b):
            return pltpu.make_async_copy(
                src_hbm.at[idx_v.at[pl.ds(c * _CHUNK, _CHUNK)]],
                rows_v.at[b], gsem[b])

        def write(c, b):
            return pltpu.make_async_copy(
                rows_v.at[b], out_hbm.at[pl.ds(base0 + c * _CHUNK, _CHUNK)],
                wsem[b])

        def body(p, _):
            for b in (0, 1):
                c = p * 2 + b

                @pl.when(p > 0)
                def _():
                    write(c, b).wait()

                gather(c, b).start()
            for b in (0, 1):
                c = p * 2 + b
                gather(c, b).wait()
                write(c, b).start()
            return ()

        lax.fori_loop(0, n_pairs, body, (), unroll=False)
        for b in (0, 1):
            write((n_pairs - 1) * 2 + b, b).wait()

    return k(src, idx)


def _mm_kernel(b2e_ref, nused_ref, x_ref, w1_ref, w2_ref, ws_ref, o_ref):
    i = pl.program_id(0)

    @pl.when(i < nused_ref[0])
    def _():
        xb = x_ref[...]
        w1 = w1_ref[0]
        h = lax.dot_general(xb, w1, (((1,), (1,)), ((), ())),
                            preferred_element_type=jnp.float32)
        f = w1.shape[0] // 2
        y = h[:, :f]
        g = h[:, f:]
        act = y * (g * jax.nn.sigmoid(g))
        w2 = w2_ref[0]
        ob = lax.dot_general(act, w2, (((1,), (1,)), ((), ())),
                             preferred_element_type=jnp.float32)
        o_ref[...] = ob * ws_ref[...]


def _grouped_mlp(x_sorted, fc1, fc2, w_sorted, b2e, num_used, num_blocks):
    p_max, d_model = x_sorted.shape
    e, two_ff, _ = fc1.shape
    d_ff = two_ff // 2

    def last_used(i, nu):
        return jnp.minimum(i, nu[0] - 1)

    grid_spec = pltpu.PrefetchScalarGridSpec(
        num_scalar_prefetch=2,
        grid=(num_blocks,),
        in_specs=[
            pl.BlockSpec((_B, d_model), lambda i, be, nu: (last_used(i, nu), 0)),
            pl.BlockSpec((1, two_ff, d_model),
                         lambda i, be, nu: (be[last_used(i, nu)], 0, 0)),
            pl.BlockSpec((1, d_model, d_ff),
                         lambda i, be, nu: (be[last_used(i, nu)], 0, 0)),
            pl.BlockSpec((_B, 1), lambda i, be, nu: (last_used(i, nu), 0)),
        ],
        out_specs=pl.BlockSpec((_B, d_model),
                               lambda i, be, nu: (last_used(i, nu), 0)),
    )
    return pl.pallas_call(
        _mm_kernel,
        grid_spec=grid_spec,
        out_shape=jax.ShapeDtypeStruct((p_max, d_model), jnp.float32),
        compiler_params=pltpu.CompilerParams(
            dimension_semantics=("arbitrary",),
            vmem_limit_bytes=100 * 1024 * 1024,
        ),
    )(b2e, num_used, x_sorted, fc1, fc2, w_sorted)


def kernel(x, weights, indices, fc1_weights, fc2_weights):
    t, d_model = x.shape
    e = fc1_weights.shape[0]
    p_max = t + e * _B
    num_blocks = t // _B + e

    flat = indices.reshape(-1).astype(jnp.int32)
    sort_ids = jnp.argsort(flat, stable=True).astype(jnp.int32)
    sorted_flat = jnp.take(flat, sort_ids)
    bounds = jnp.searchsorted(
        sorted_flat, jnp.arange(e + 1, dtype=jnp.int32), side="left"
    ).astype(jnp.int32)
    counts = bounds[1:] - bounds[:-1]
    nblk = (counts + _B - 1) // _B
    starts = jnp.concatenate(
        [jnp.zeros((1,), jnp.int32), jnp.cumsum(nblk * _B).astype(jnp.int32)]
    )
    num_used = (starts[e] // _B).reshape(1).astype(jnp.int32)

    p = jnp.arange(p_max, dtype=jnp.int32)
    e_of_p = jnp.minimum(
        jnp.searchsorted(starts[1:], p, side="right").astype(jnp.int32), e - 1)
    local = p - jnp.take(starts, e_of_p)
    r = jnp.take(bounds, e_of_p) + local
    valid = local < jnp.take(counts, e_of_p)
    src = jnp.where(valid, jnp.take(sort_ids, jnp.clip(r, 0, t - 1)),
                    p & (t - 1))
    src = src.astype(jnp.int32)

    inv_rank = jnp.argsort(sort_ids).astype(jnp.int32)
    pos = (jnp.take(starts, flat) + (inv_rank - jnp.take(bounds, flat))
           ).astype(jnp.int32)

    blk_p = jnp.arange(num_blocks, dtype=jnp.int32) * _B
    b2e = jnp.minimum(
        jnp.searchsorted(starts[1:], blk_p, side="right").astype(jnp.int32),
        e - 1)

    w_sorted = jnp.take(weights[:, 0], src).reshape(p_max, 1)

    x_sorted = _sc_row_gather(x, src, p_max)
    y_sorted = _grouped_mlp(x_sorted, fc1_weights, fc2_weights, w_sorted,
                            b2e, num_used, num_blocks)
    out = _sc_row_gather(y_sorted, pos, t)
    return out


# speedup vs baseline: 3.5263x; 1.1222x over previous
import functools

import jax
import jax.numpy as jnp
from jax import lax
from jax.experimental import pallas as pl
from jax.experimental.pallas import tpu as pltpu
from jax.experimental.pallas import tpu_sc as plsc

_B = 128
_CHUNK = 16


def _sc_row_gather(src, idx, n_out):
    info = plsc.get_sparse_core_info()
    nw = info.num_cores * info.num_subcores
    d = src.shape[1]
    assert n_out % (nw * 2 * _CHUNK) == 0
    per_w = n_out // nw
    n_pairs = per_w // (2 * _CHUNK)

    mesh = plsc.VectorSubcoreMesh(core_axis_name="c", subcore_axis_name="s")

    @functools.partial(
        pl.kernel,
        mesh=mesh,
        out_type=jax.ShapeDtypeStruct((n_out, d), src.dtype),
        scratch_types=[
            pltpu.VMEM((per_w,), jnp.int32),
            pltpu.VMEM((2, _CHUNK, d), src.dtype),
            pltpu.SemaphoreType.DMA,
            pltpu.SemaphoreType.DMA,
            pltpu.SemaphoreType.DMA,
            pltpu.SemaphoreType.DMA,
        ],
    )
    def k(src_hbm, idx_hbm, out_hbm, idx_v, rows_v, g0, g1, w0, w1):
        wid = lax.axis_index("s") * info.num_cores + lax.axis_index("c")
        base0 = wid * per_w
        pltpu.sync_copy(idx_hbm.at[pl.ds(base0, per_w)], idx_v)
        gsem = (g0, g1)
        wsem = (w0, w1)

        def gather(c, b):
            return pltpu.make_async_copy(
                src_hbm.at[idx_v.at[pl.ds(c * _CHUNK, _CHUNK)]],
                rows_v.at[b], gsem[b])

        def write(c, b):
            return pltpu.make_async_copy(
                rows_v.at[b], out_hbm.at[pl.ds(base0 + c * _CHUNK, _CHUNK)],
                wsem[b])

        def body(p, _):
            for b in (0, 1):
                c = p * 2 + b

                @pl.when(p > 0)
                def _():
                    write(c, b).wait()

                gather(c, b).start()
            for b in (0, 1):
                c = p * 2 + b
                gather(c, b).wait()
                write(c, b).start()
            return ()

        lax.fori_loop(0, n_pairs, body, (), unroll=False)
        for b in (0, 1):
            write((n_pairs - 1) * 2 + b, b).wait()

    return k(src, idx)


def _mm_kernel(b2e_ref, nused_ref, x_ref, w1_ref, w2_ref, ws_ref, o_ref):
    i = pl.program_id(0)

    @pl.when(i < nused_ref[0])
    def _():
        xb = x_ref[...]
        w1 = w1_ref[0]
        h = lax.dot_general(xb, w1, (((1,), (1,)), ((), ())),
                            preferred_element_type=jnp.float32)
        f = w1.shape[0] // 2
        y = h[:, :f]
        g = h[:, f:]
        act = y * (g * jax.nn.sigmoid(g))
        w2 = w2_ref[0]
        ob = lax.dot_general(act, w2, (((1,), (1,)), ((), ())),
                             preferred_element_type=jnp.float32)
        o_ref[...] = ob * ws_ref[...]


def _grouped_mlp(x_sorted, fc1, fc2, w_sorted, b2e, num_used, num_blocks):
    p_max, d_model = x_sorted.shape
    e, two_ff, _ = fc1.shape
    d_ff = two_ff // 2

    def last_used(i, nu):
        return jnp.minimum(i, nu[0] - 1)

    grid_spec = pltpu.PrefetchScalarGridSpec(
        num_scalar_prefetch=2,
        grid=(num_blocks,),
        in_specs=[
            pl.BlockSpec((_B, d_model), lambda i, be, nu: (last_used(i, nu), 0)),
            pl.BlockSpec((1, two_ff, d_model),
                         lambda i, be, nu: (be[last_used(i, nu)], 0, 0)),
            pl.BlockSpec((1, d_model, d_ff),
                         lambda i, be, nu: (be[last_used(i, nu)], 0, 0)),
            pl.BlockSpec((_B, 1), lambda i, be, nu: (last_used(i, nu), 0)),
        ],
        out_specs=pl.BlockSpec((_B, d_model),
                               lambda i, be, nu: (last_used(i, nu), 0)),
    )
    return pl.pallas_call(
        _mm_kernel,
        grid_spec=grid_spec,
        out_shape=jax.ShapeDtypeStruct((p_max, d_model), jnp.float32),
        compiler_params=pltpu.CompilerParams(
            dimension_semantics=("arbitrary",),
            vmem_limit_bytes=100 * 1024 * 1024,
        ),
    )(b2e, num_used, x_sorted, fc1, fc2, w_sorted)


def kernel(x, weights, indices, fc1_weights, fc2_weights):
    t, d_model = x.shape
    e = fc1_weights.shape[0]
    p_max = t + e * _B
    num_blocks = t // _B + e

    flat = indices.reshape(-1).astype(jnp.int32)
    onehot = (flat[:, None] == jnp.arange(e, dtype=jnp.int32)[None, :])
    csum = jnp.cumsum(onehot.astype(jnp.int32), axis=0)
    counts = csum[-1]
    rank = jnp.take_along_axis(csum, flat[:, None], axis=1)[:, 0] - 1
    nblk = (counts + _B - 1) // _B
    starts = jnp.concatenate(
        [jnp.zeros((1,), jnp.int32), jnp.cumsum(nblk * _B).astype(jnp.int32)]
    )
    num_used = (starts[e] // _B).reshape(1).astype(jnp.int32)

    pos = (jnp.take(starts, flat) + rank).astype(jnp.int32)
    p = jnp.arange(p_max, dtype=jnp.int32)
    src = (p & (t - 1)).at[pos].set(jnp.arange(t, dtype=jnp.int32))

    blk_p = jnp.arange(num_blocks, dtype=jnp.int32) * _B
    b2e = jnp.minimum(
        jnp.searchsorted(starts[1:], blk_p, side="right").astype(jnp.int32),
        e - 1)

    w_sorted = jnp.zeros((p_max,), jnp.float32).at[pos].set(
        weights[:, 0]).reshape(p_max, 1)

    x_sorted = _sc_row_gather(x, src, p_max)
    y_sorted = _grouped_mlp(x_sorted, fc1_weights, fc2_weights, w_sorted,
                            b2e, num_used, num_blocks)
    out = _sc_row_gather(y_sorted, pos, t)
    return out


# speedup vs baseline: 4.9288x; 1.3977x over previous
import functools

import jax
import jax.numpy as jnp
from jax import lax
from jax.experimental import pallas as pl
from jax.experimental.pallas import tpu as pltpu
from jax.experimental.pallas import tpu_sc as plsc

_B = 256
_CHUNK = 16


def _sc_row_gather(src, idx, n_out):
    info = plsc.get_sparse_core_info()
    nw = info.num_cores * info.num_subcores
    d = src.shape[1]
    assert n_out % (nw * 2 * _CHUNK) == 0
    per_w = n_out // nw
    n_pairs = per_w // (2 * _CHUNK)

    mesh = plsc.VectorSubcoreMesh(core_axis_name="c", subcore_axis_name="s")

    @functools.partial(
        pl.kernel,
        mesh=mesh,
        out_type=jax.ShapeDtypeStruct((n_out, d), src.dtype),
        scratch_types=[
            pltpu.VMEM((per_w,), jnp.int32),
            pltpu.VMEM((2, _CHUNK, d), src.dtype),
            pltpu.SemaphoreType.DMA,
            pltpu.SemaphoreType.DMA,
            pltpu.SemaphoreType.DMA,
            pltpu.SemaphoreType.DMA,
        ],
    )
    def k(src_hbm, idx_hbm, out_hbm, idx_v, rows_v, g0, g1, w0, w1):
        wid = lax.axis_index("s") * info.num_cores + lax.axis_index("c")
        base0 = wid * per_w
        pltpu.sync_copy(idx_hbm.at[pl.ds(base0, per_w)], idx_v)
        gsem = (g0, g1)
        wsem = (w0, w1)

        def gather(c, b):
            return pltpu.make_async_copy(
                src_hbm.at[idx_v.at[pl.ds(c * _CHUNK, _CHUNK)]],
                rows_v.at[b], gsem[b])

        def write(c, b):
            return pltpu.make_async_copy(
                rows_v.at[b], out_hbm.at[pl.ds(base0 + c * _CHUNK, _CHUNK)],
                wsem[b])

        def body(p, _):
            for b in (0, 1):
                c = p * 2 + b

                @pl.when(p > 0)
                def _():
                    write(c, b).wait()

                gather(c, b).start()
            for b in (0, 1):
                c = p * 2 + b
                gather(c, b).wait()
                write(c, b).start()
            return ()

        lax.fori_loop(0, n_pairs, body, (), unroll=False)
        for b in (0, 1):
            write((n_pairs - 1) * 2 + b, b).wait()

    return k(src, idx)


def _mm_kernel(b2e_ref, nused_ref, x_ref, w1_ref, w2_ref, ws_ref, o_ref):
    i = pl.program_id(0)

    @pl.when(i < nused_ref[0])
    def _():
        xb = x_ref[...]
        w1 = w1_ref[0]
        h = lax.dot_general(xb, w1, (((1,), (1,)), ((), ())),
                            preferred_element_type=jnp.float32)
        f = w1.shape[0] // 2
        y = h[:, :f]
        g = h[:, f:]
        act = y * (g * jax.nn.sigmoid(g))
        w2 = w2_ref[0]
        ob = lax.dot_general(act, w2, (((1,), (1,)), ((), ())),
                             preferred_element_type=jnp.float32)
        o_ref[...] = ob * ws_ref[...]


def _grouped_mlp(x_sorted, fc1, fc2, w_sorted, b2e, num_used, num_blocks):
    p_max, d_model = x_sorted.shape
    e, two_ff, _ = fc1.shape
    d_ff = two_ff // 2

    def last_used(i, nu):
        return jnp.minimum(i, nu[0] - 1)

    grid_spec = pltpu.PrefetchScalarGridSpec(
        num_scalar_prefetch=2,
        grid=(num_blocks,),
        in_specs=[
            pl.BlockSpec((_B, d_model), lambda i, be, nu: (last_used(i, nu), 0)),
            pl.BlockSpec((1, two_ff, d_model),
                         lambda i, be, nu: (be[last_used(i, nu)], 0, 0)),
            pl.BlockSpec((1, d_model, d_ff),
                         lambda i, be, nu: (be[last_used(i, nu)], 0, 0)),
            pl.BlockSpec((_B, 1), lambda i, be, nu: (last_used(i, nu), 0)),
        ],
        out_specs=pl.BlockSpec((_B, d_model),
                               lambda i, be, nu: (last_used(i, nu), 0)),
    )
    return pl.pallas_call(
        _mm_kernel,
        grid_spec=grid_spec,
        out_shape=jax.ShapeDtypeStruct((p_max, d_model), jnp.float32),
        compiler_params=pltpu.CompilerParams(
            dimension_semantics=("arbitrary",),
            vmem_limit_bytes=100 * 1024 * 1024,
        ),
    )(b2e, num_used, x_sorted, fc1, fc2, w_sorted)


def kernel(x, weights, indices, fc1_weights, fc2_weights):
    t, d_model = x.shape
    e = fc1_weights.shape[0]
    p_max = t + e * _B
    num_blocks = t // _B + e

    flat = indices.reshape(-1).astype(jnp.int32)
    onehot = (flat[:, None] == jnp.arange(e, dtype=jnp.int32)[None, :])
    csum = jnp.cumsum(onehot.astype(jnp.int32), axis=0)
    counts = csum[-1]
    rank = jnp.take_along_axis(csum, flat[:, None], axis=1)[:, 0] - 1
    nblk = (counts + _B - 1) // _B
    starts = jnp.concatenate(
        [jnp.zeros((1,), jnp.int32), jnp.cumsum(nblk * _B).astype(jnp.int32)]
    )
    num_used = (starts[e] // _B).reshape(1).astype(jnp.int32)

    pos = (jnp.take(starts, flat) + rank).astype(jnp.int32)
    p = jnp.arange(p_max, dtype=jnp.int32)
    src = (p & (t - 1)).at[pos].set(jnp.arange(t, dtype=jnp.int32))

    blk_p = jnp.arange(num_blocks, dtype=jnp.int32) * _B
    b2e = jnp.minimum(
        jnp.searchsorted(starts[1:], blk_p, side="right").astype(jnp.int32),
        e - 1)

    w_sorted = jnp.zeros((p_max,), jnp.float32).at[pos].set(
        weights[:, 0]).reshape(p_max, 1)

    x_sorted = _sc_row_gather(x, src, p_max)
    y_sorted = _grouped_mlp(x_sorted, fc1_weights, fc2_weights, w_sorted,
                            b2e, num_used, num_blocks)
    out = _sc_row_gather(y_sorted, pos, t)
    return out


# speedup vs baseline: 4.9490x; 1.0041x over previous
import functools

import jax
import jax.numpy as jnp
from jax import lax
from jax.experimental import pallas as pl
from jax.experimental.pallas import tpu as pltpu
from jax.experimental.pallas import tpu_sc as plsc

_B = 256
_CHUNK = 16


def _sc_row_gather(src, idx, n_out):
    info = plsc.get_sparse_core_info()
    nw = info.num_cores * info.num_subcores
    d = src.shape[1]
    assert n_out % (nw * 2 * _CHUNK) == 0
    per_w = n_out // nw
    n_pairs = per_w // (2 * _CHUNK)

    mesh = plsc.VectorSubcoreMesh(core_axis_name="c", subcore_axis_name="s")

    @functools.partial(
        pl.kernel,
        mesh=mesh,
        out_type=jax.ShapeDtypeStruct((n_out, d), src.dtype),
        scratch_types=[
            pltpu.VMEM((per_w,), jnp.int32),
            pltpu.VMEM((2, _CHUNK, d), src.dtype),
            pltpu.SemaphoreType.DMA,
            pltpu.SemaphoreType.DMA,
            pltpu.SemaphoreType.DMA,
            pltpu.SemaphoreType.DMA,
        ],
    )
    def k(src_hbm, idx_hbm, out_hbm, idx_v, rows_v, g0, g1, w0, w1):
        wid = lax.axis_index("s") * info.num_cores + lax.axis_index("c")
        base0 = wid * per_w
        pltpu.sync_copy(idx_hbm.at[pl.ds(base0, per_w)], idx_v)
        gsem = (g0, g1)
        wsem = (w0, w1)

        def gather(c, b):
            return pltpu.make_async_copy(
                src_hbm.at[idx_v.at[pl.ds(c * _CHUNK, _CHUNK)]],
                rows_v.at[b], gsem[b])

        def write(c, b):
            return pltpu.make_async_copy(
                rows_v.at[b], out_hbm.at[pl.ds(base0 + c * _CHUNK, _CHUNK)],
                wsem[b])

        def body(p, _):
            for b in (0, 1):
                c = p * 2 + b

                @pl.when(p > 0)
                def _():
                    write(c, b).wait()

                gather(c, b).start()
            for b in (0, 1):
                c = p * 2 + b
                gather(c, b).wait()
                write(c, b).start()
            return ()

        lax.fori_loop(0, n_pairs, body, (), unroll=False)
        for b in (0, 1):
            write((n_pairs - 1) * 2 + b, b).wait()

    return k(src, idx)


def _mm_kernel(b2e_ref, nused_ref, x_ref, w1_ref, w2_ref, ws_ref, o_ref):
    i = pl.program_id(0)

    @pl.when(i < nused_ref[0])
    def _():
        xb = x_ref[...]
        w1 = w1_ref[0]
        h = lax.dot_general(xb, w1, (((1,), (1,)), ((), ())),
                            preferred_element_type=jnp.float32)
        f = w1.shape[0] // 2
        y = h[:, :f]
        g = h[:, f:]
        act = y * (g * jax.nn.sigmoid(g))
        w2 = w2_ref[0]
        ob = lax.dot_general(act, w2, (((1,), (1,)), ((), ())),
                             preferred_element_type=jnp.float32)
        off = jnp.minimum(i, nused_ref[0] - 1) * x_ref.shape[0]
        o_ref[...] = ob * ws_ref[pl.ds(off, x_ref.shape[0]), :]


def _grouped_mlp(x_sorted, fc1, fc2, w_sorted, b2e, num_used, num_blocks):
    p_max, d_model = x_sorted.shape
    e, two_ff, _ = fc1.shape
    d_ff = two_ff // 2

    def last_used(i, nu):
        return jnp.minimum(i, nu[0] - 1)

    grid_spec = pltpu.PrefetchScalarGridSpec(
        num_scalar_prefetch=2,
        grid=(num_blocks,),
        in_specs=[
            pl.BlockSpec((_B, d_model), lambda i, be, nu: (last_used(i, nu), 0)),
            pl.BlockSpec((1, two_ff, d_model),
                         lambda i, be, nu: (be[last_used(i, nu)], 0, 0)),
            pl.BlockSpec((1, d_model, d_ff),
                         lambda i, be, nu: (be[last_used(i, nu)], 0, 0)),
            pl.BlockSpec((p_max, 1), lambda i, be, nu: (0, 0)),
        ],
        out_specs=pl.BlockSpec((_B, d_model),
                               lambda i, be, nu: (last_used(i, nu), 0)),
    )
    return pl.pallas_call(
        _mm_kernel,
        grid_spec=grid_spec,
        out_shape=jax.ShapeDtypeStruct((p_max, d_model), jnp.float32),
        compiler_params=pltpu.CompilerParams(
            dimension_semantics=("arbitrary",),
            vmem_limit_bytes=100 * 1024 * 1024,
        ),
    )(b2e, num_used, x_sorted, fc1, fc2, w_sorted)


def kernel(x, weights, indices, fc1_weights, fc2_weights):
    t, d_model = x.shape
    e = fc1_weights.shape[0]
    p_max = t + e * _B
    num_blocks = t // _B + e

    flat = indices.reshape(-1).astype(jnp.int32)
    onehot = (flat[:, None] == jnp.arange(e, dtype=jnp.int32)[None, :])
    csum = jnp.cumsum(onehot.astype(jnp.int32), axis=0)
    counts = csum[-1]
    rank = jnp.sum(csum * onehot, axis=1) - 1
    nblk = (counts + _B - 1) // _B
    starts = jnp.concatenate(
        [jnp.zeros((1,), jnp.int32), jnp.cumsum(nblk * _B).astype(jnp.int32)]
    )
    num_used = (starts[e] // _B).reshape(1).astype(jnp.int32)

    pos = (jnp.take(starts, flat) + rank).astype(jnp.int32)
    p = jnp.arange(p_max, dtype=jnp.int32)
    src = (p & (t - 1)).at[pos].set(jnp.arange(t, dtype=jnp.int32))

    blk_p = jnp.arange(num_blocks, dtype=jnp.int32) * _B
    b2e = jnp.minimum(
        jnp.searchsorted(starts[1:], blk_p, side="right").astype(jnp.int32),
        e - 1)

    w_sorted = jnp.zeros((p_max,), jnp.float32).at[pos].set(
        weights[:, 0]).reshape(p_max, 1)

    x_sorted = _sc_row_gather(x, src, p_max)
    y_sorted = _grouped_mlp(x_sorted, fc1_weights, fc2_weights, w_sorted,
                            b2e, num_used, num_blocks)
    out = _sc_row_gather(y_sorted, pos, t)
    return out
